# Initial kernel scaffold; baseline (speedup 1.0000x reference)
import jax
import jax.numpy as jnp
from jax.experimental import pallas as pl


def kernel(z, proto_embs):
    raise NotImplementedError("write your pallas kernel here")


# speedup vs baseline: 12.8149x; 12.8149x over previous
import jax
import jax.numpy as jnp
from jax.experimental import pallas as pl
from jax.experimental.pallas import tpu as pltpu

_TOPK = 8
_TEMP = 0.1
_EPS = 1e-07


def _fused_body(z_ref, p_ref, sim_ref, out_ref):
    z = z_ref[...]
    p = p_ref[...]
    zn = z / (jnp.sqrt(jnp.sum(z * z, axis=-1, keepdims=True)) + _EPS)
    pn = p / (jnp.sqrt(jnp.sum(p * p, axis=-1, keepdims=True)) + _EPS)
    sim = jax.lax.dot_general(
        zn, pn, (((1,), (1,)), ((), ())),
        preferred_element_type=jnp.float32,
    ) * (1.0 / _TEMP)
    sim_ref[...] = sim

    run = sim
    m1 = jnp.max(run, axis=-1, keepdims=True)
    cur = m1
    for _ in range(_TOPK - 1):
        run = jnp.where(run >= cur, -jnp.inf, run)
        cur = jnp.max(run, axis=-1, keepdims=True)
    t8 = cur

    w = jnp.where(sim >= t8, jnp.exp(sim - m1), 0.0)
    denom = jnp.sum(w, axis=-1, keepdims=True)
    comb = jax.lax.dot_general(
        w, p, (((1,), (0,)), ((), ())),
        preferred_element_type=jnp.float32,
    )
    out_ref[...] = comb / denom


def _pick_block(total: int, target: int = 1024) -> int:
    best = 1
    for t in range(8, target + 1, 8):
        if total % t == 0:
            best = t
    return best if best > 1 else total


def kernel(z, proto_embs):
    b, l, d = z.shape
    k = proto_embs.shape[0]
    total = b * l
    t_blk = _pick_block(total)
    grid = (total // t_blk,)

    zf = z.reshape(total, d)
    sim_flat, out_flat = pl.pallas_call(
        _fused_body,
        grid=grid,
        in_specs=[
            pl.BlockSpec((t_blk, d), lambda i: (i, 0)),
            pl.BlockSpec((k, d), lambda i: (0, 0)),
        ],
        out_specs=[
            pl.BlockSpec((t_blk, k), lambda i: (i, 0)),
            pl.BlockSpec((t_blk, d), lambda i: (i, 0)),
        ],
        out_shape=[
            jax.ShapeDtypeStruct((total, k), jnp.float32),
            jax.ShapeDtypeStruct((total, d), jnp.float32),
        ],
        compiler_params=pltpu.CompilerParams(
            dimension_semantics=("parallel",),
        ),
    )(zf, proto_embs)
    return out_flat.reshape(b, l, d), sim_flat.reshape(b, l, k)
